# baseline (speedup 1.0000x reference)
import jax
import jax.numpy as jnp
from jax.experimental import pallas as pl
from jax.experimental.pallas import tpu as pltpu

_NB = 8
_K5 = 5
_P1 = 208
_P2 = 80


def _wshift(v):
    return jnp.concatenate([v[:, 1:], v[:, :1]], axis=1)


def _cnn_body(x_ref, t1_ref, t2_ref, t3_ref, fw1_ref, s1_ref, s2_ref, s3_ref,
              bias1_ref, bias2_ref, bias3_ref, fb1_ref, fw2_ref, fb2_ref,
              out_ref,
              x5_ref, a1_ref, e1_ref, p1s_ref, a2_ref, e2_ref, p2s_ref,
              a3_ref, e3_ref):
    nb = _NB
    f32 = jnp.float32
    bf16 = jnp.bfloat16

    xv = x_ref[...].astype(bf16)
    l1 = nb * 50 - 4
    for kh in range(_K5):
        x5_ref[pl.ds(kh * nb * 50, l1), :] = xv[kh:kh + l1]

    acc = jnp.dot(x5_ref[pl.ds(0, l1), :], t1_ref[pl.ds(0, 50), :],
                  preferred_element_type=f32)
    for kh in range(1, _K5):
        acc += jnp.dot(x5_ref[pl.ds(kh * nb * 50, l1), :],
                       t1_ref[pl.ds(kh * 50, 50), :],
                       preferred_element_type=f32)
    a1_ref[pl.ds(0, l1), :] = jnp.maximum(acc + bias1_ref[...], 0.0)
    a1_ref[pl.ds(l1, 4), :] = jnp.zeros((4, 1472), f32)

    e1 = jnp.maximum(a1_ref[pl.ds(0, nb * 50 - 1), :],
                     a1_ref[pl.ds(1, nb * 50 - 1), :])
    e1_ref[pl.ds(0, nb * 50 - 1), :] = jnp.maximum(e1, _wshift(e1)).astype(bf16)
    e1_ref[pl.ds(nb * 50 - 1, 1), :] = jnp.zeros((1, 1472), bf16)
    p1s_ref[...] = jnp.dot(s1_ref[...], e1_ref[...],
                           preferred_element_type=f32).astype(bf16)

    l2 = nb * 25 - 4
    acc = jnp.dot(p1s_ref[pl.ds(0, l2), :], t2_ref[pl.ds(0, 1472), :],
                  preferred_element_type=f32)
    for kh in range(1, _K5):
        acc += jnp.dot(p1s_ref[pl.ds(kh * _P1, l2), :],
                       t2_ref[pl.ds(kh * 1472, 1472), :],
                       preferred_element_type=f32)
    a2_ref[pl.ds(0, l2), :] = jnp.maximum(acc + bias2_ref[...], 0.0)
    a2_ref[pl.ds(l2, 4), :] = jnp.zeros((4, 1152), f32)

    e2 = jnp.maximum(a2_ref[pl.ds(0, nb * 25 - 1), :],
                     a2_ref[pl.ds(1, nb * 25 - 1), :])
    e2_ref[pl.ds(0, nb * 25 - 1), :] = jnp.maximum(e2, _wshift(e2)).astype(bf16)
    e2_ref[pl.ds(nb * 25 - 1, 1), :] = jnp.zeros((1, 1152), bf16)
    p2s_ref[...] = jnp.dot(s2_ref[...], e2_ref[...],
                           preferred_element_type=f32).astype(bf16)

    l3 = nb * 9 - 4
    acc = jnp.dot(p2s_ref[pl.ds(0, l3), :], t3_ref[pl.ds(0, 1152), :],
                  preferred_element_type=f32)
    for kh in range(1, _K5):
        acc += jnp.dot(p2s_ref[pl.ds(kh * _P2, l3), :],
                       t3_ref[pl.ds(kh * 1152, 1152), :],
                       preferred_element_type=f32)
    a3_ref[pl.ds(0, l3), :] = jnp.maximum(acc + bias3_ref[...], 0.0)
    a3_ref[pl.ds(l3, 4), :] = jnp.zeros((4, 512), f32)

    e3 = jnp.maximum(a3_ref[pl.ds(0, nb * 9 - 1), :],
                     a3_ref[pl.ds(1, nb * 9 - 1), :])
    e3_ref[pl.ds(0, nb * 9 - 1), :] = jnp.maximum(e3, _wshift(e3)).astype(bf16)
    e3_ref[pl.ds(nb * 9 - 1, 1), :] = jnp.zeros((1, 512), bf16)
    c3 = jnp.dot(s3_ref[...], e3_ref[...], preferred_element_type=f32)
    feats = jnp.concatenate([c3[0:nb], c3[nb:2 * nb]], axis=1).astype(bf16)

    h = jnp.dot(feats, fw1_ref[...], preferred_element_type=f32)
    h = jnp.maximum(h + fb1_ref[...], 0.0)
    logits = jnp.dot(h, fw2_ref[...], preferred_element_type=f32) + fb2_ref[...]
    m = jnp.max(logits, axis=-1, keepdims=True)
    ex = jnp.exp(logits - m)
    out_ref[...] = ex / jnp.sum(ex, axis=-1, keepdims=True)


def _toeplitz(w, hi, wo):
    kw = w.shape[1]
    shift = jnp.stack([jnp.eye(hi, wo, k=-x, dtype=w.dtype) for x in range(kw)])
    return jnp.einsum('xvw,hxcd->hcvdw', shift, w)


def _interleave_zeros(t):
    k, c, hp, d, w = t.shape
    z = jnp.zeros((k, c, 2 * hp, d, w), t.dtype)
    return z.at[:, :, 0::2].set(t)


def _rowsel_shifted(nb, pitch_in, pitch_out, block, n_valid):
    n_src = nb * pitch_in
    blocks = []
    for kh in range(_K5):
        r = jnp.arange(block)
        rp = r + kh
        src = (rp // pitch_out) * pitch_in + 2 * (rp % pitch_out)
        valid = r < n_valid
        m = jnp.zeros((block, n_src), jnp.float32)
        m = m.at[r, jnp.minimum(src, n_src - 1)].set(
            jnp.where(valid, 1.0, 0.0))
        blocks.append(m)
    return jnp.concatenate(blocks, axis=0)


@jax.jit
def _forward(x_nchw, w1m, b1, w2m, b2, w3m, b3, fw1, fb1, fw2, fb2):
    f32 = jnp.float32
    bf16 = jnp.bfloat16
    bsz = x_nchw.shape[0]
    nb = _NB

    w1r = w1m.reshape(5, 5, 1, 32)
    w2r = w2m.reshape(5, 5, 32, 64)
    w3r = w3m.reshape(5, 5, 64, 128)

    t1 = _toeplitz(w1r, 50, 46).reshape(5 * 50, 32 * 46).astype(bf16)
    t2 = _interleave_zeros(_toeplitz(w2r, 23, 18))
    t2 = t2.reshape(5 * 1472, 64 * 18).astype(bf16)
    t3 = _interleave_zeros(_toeplitz(w3r, 9, 4))
    t3 = t3.reshape(5 * 1152, 128 * 4).astype(bf16)

    fw1v = fw1.reshape(2, 2, 128, 512)
    fw1v = jnp.transpose(fw1v, (0, 2, 1, 3))
    fw1p = jnp.zeros((2, 128, 4, 512), fw1.dtype).at[:, :, 0::2, :].set(fw1v)
    fw1p = fw1p.reshape(1024, 512).astype(bf16)

    s1 = _rowsel_shifted(nb, 50, 25, _P1, nb * 25 - 4).astype(bf16)
    s2 = _rowsel_shifted(nb, 25, 9, _P2, nb * 9 - 4).astype(bf16)
    e0 = jnp.zeros((1, 9), f32).at[0, 0].set(1.0)
    e2v = jnp.zeros((1, 9), f32).at[0, 2].set(1.0)
    s3 = jnp.concatenate([jnp.kron(jnp.eye(nb, dtype=f32), e0),
                          jnp.kron(jnp.eye(nb, dtype=f32), e2v)], axis=0)
    s3 = s3.astype(bf16)

    bias1 = jnp.repeat(b1.reshape(32, 1), 46, axis=1).reshape(1, 1472)
    bias2 = jnp.repeat(b2.reshape(64, 1), 18, axis=1).reshape(1, 1152)
    bias3 = jnp.repeat(b3.reshape(128, 1), 4, axis=1).reshape(1, 512)

    bpad = (-bsz) % nb
    x = x_nchw.astype(f32).reshape(bsz, 50, 50)
    if bpad:
        x = jnp.concatenate([x, jnp.zeros((bpad, 50, 50), f32)], axis=0)
    bt = bsz + bpad
    x = x.reshape(bt * 50, 50)

    const = lambda i: (0, 0)
    in_specs = [
        pl.BlockSpec((nb * 50, 50), lambda i: (i, 0)),
        pl.BlockSpec((5 * 50, 1472), const),
        pl.BlockSpec((5 * 1472, 1152), const),
        pl.BlockSpec((5 * 1152, 512), const),
        pl.BlockSpec((1024, 512), const),
        pl.BlockSpec((5 * _P1, nb * 50), const),
        pl.BlockSpec((5 * _P2, nb * 25), const),
        pl.BlockSpec((2 * nb, nb * 9), const),
        pl.BlockSpec((1, 1472), const),
        pl.BlockSpec((1, 1152), const),
        pl.BlockSpec((1, 512), const),
        pl.BlockSpec((1, 512), const),
        pl.BlockSpec((512, 2), const),
        pl.BlockSpec((1, 2), const),
    ]
    scratch_shapes = [
        pltpu.VMEM((5 * nb * 50, 50), bf16),
        pltpu.VMEM((nb * 50, 1472), f32),
        pltpu.VMEM((nb * 50, 1472), bf16),
        pltpu.VMEM((5 * _P1, 1472), bf16),
        pltpu.VMEM((nb * 25, 1152), f32),
        pltpu.VMEM((nb * 25, 1152), bf16),
        pltpu.VMEM((5 * _P2, 1152), bf16),
        pltpu.VMEM((nb * 9, 512), f32),
        pltpu.VMEM((nb * 9, 512), bf16),
    ]
    cls = getattr(pltpu, "CompilerParams", None) or getattr(
        pltpu, "TPUCompilerParams", None)
    cparams = None
    if cls is not None:
        cparams = cls(dimension_semantics=("parallel",),
                      vmem_limit_bytes=56 * 1024 * 1024)

    out = pl.pallas_call(
        _cnn_body,
        out_shape=jax.ShapeDtypeStruct((bt, 2), f32),
        grid=(bt // nb,),
        in_specs=in_specs,
        out_specs=pl.BlockSpec((nb, 2), lambda i: (i, 0)),
        scratch_shapes=scratch_shapes,
        compiler_params=cparams,
    )(x, t1, t2, t3, fw1p, s1, s2, s3, bias1, bias2, bias3, fb1, fw2, fb2)
    return out[:bsz]


def kernel(x_nchw, w1m, b1, w2m, b2, w3m, b3, fw1, fb1, fw2, fb2):
    return _forward(x_nchw, w1m, b1, w2m, b2, w3m, b3, fw1, fb1, fw2, fb2)


# speedup vs baseline: 1.6483x; 1.6483x over previous
import jax
import jax.numpy as jnp
from jax.experimental import pallas as pl
from jax.experimental.pallas import tpu as pltpu

_NB = 16
_K5 = 5


def _wshift(v):
    return jnp.concatenate([v[:, 1:], v[:, :1]], axis=1)


def _cnn_body(x_ref, t1_ref, t2_ref, t3_ref, fw1_ref, s1_ref, s2_ref, s3_ref,
              c1_ref, c2_ref, c3m_ref,
              bias1_ref, bias2_ref, bias3_ref, fb1_ref, fw2_ref, fb2_ref,
              out_ref,
              x1p_ref, a1_ref, e1_ref, q1_ref, p1p_ref, a2_ref, e2_ref,
              q2_ref, p2p_ref, a3_ref, e3_ref):
    nb = _NB
    f32 = jnp.float32
    bf16 = jnp.bfloat16

    xv = x_ref[...].astype(bf16)
    l1 = nb * 50 - 4
    for kh in range(_K5):
        x1p_ref[pl.ds(0, l1), pl.ds(kh * 64, 50)] = xv[kh:kh + l1]
        x1p_ref[:, pl.ds(kh * 64 + 50, 14)] = jnp.zeros((nb * 50, 14), bf16)

    acc = jnp.dot(x1p_ref[pl.ds(0, l1), :], t1_ref[...],
                  preferred_element_type=f32)
    a1_ref[pl.ds(0, l1), :] = jnp.maximum(acc + bias1_ref[...], 0.0)
    a1_ref[pl.ds(l1, 4), :] = jnp.zeros((4, 1472), f32)

    e1 = jnp.maximum(a1_ref[pl.ds(0, nb * 50 - 1), :],
                     a1_ref[pl.ds(1, nb * 50 - 1), :])
    e1_ref[pl.ds(0, nb * 50 - 1), :] = jnp.maximum(e1, _wshift(e1)).astype(bf16)
    e1_ref[pl.ds(nb * 50 - 1, 1), :] = jnp.zeros((1, 1472), bf16)
    q1_ref[...] = jnp.dot(s1_ref[...], e1_ref[...],
                          preferred_element_type=f32).astype(bf16)
    p1 = jnp.dot(q1_ref[...], c1_ref[...],
                 preferred_element_type=f32).astype(bf16)

    l2 = nb * 25 - 4
    for kh in range(_K5):
        p1p_ref[pl.ds(0, l2), pl.ds(kh * 768, 736)] = p1[kh:kh + l2]
        p1p_ref[:, pl.ds(kh * 768 + 736, 32)] = jnp.zeros((nb * 25, 32), bf16)

    acc = jnp.dot(p1p_ref[pl.ds(0, l2), :], t2_ref[...],
                  preferred_element_type=f32)
    a2_ref[pl.ds(0, l2), :] = jnp.maximum(acc + bias2_ref[...], 0.0)
    a2_ref[pl.ds(l2, 4), :] = jnp.zeros((4, 1152), f32)

    e2 = jnp.maximum(a2_ref[pl.ds(0, nb * 25 - 1), :],
                     a2_ref[pl.ds(1, nb * 25 - 1), :])
    e2_ref[pl.ds(0, nb * 25 - 1), :] = jnp.maximum(e2, _wshift(e2)).astype(bf16)
    e2_ref[pl.ds(nb * 25 - 1, 1), :] = jnp.zeros((1, 1152), bf16)
    q2_ref[...] = jnp.dot(s2_ref[...], e2_ref[...],
                          preferred_element_type=f32).astype(bf16)
    p2 = jnp.dot(q2_ref[...], c2_ref[...],
                 preferred_element_type=f32).astype(bf16)

    l3 = nb * 9 - 4
    for kh in range(_K5):
        p2p_ref[pl.ds(0, l3), pl.ds(kh * 640, 576)] = p2[kh:kh + l3]
        p2p_ref[:, pl.ds(kh * 640 + 576, 64)] = jnp.zeros((nb * 9, 64), bf16)

    acc = jnp.dot(p2p_ref[pl.ds(0, l3), :], t3_ref[...],
                  preferred_element_type=f32)
    a3_ref[pl.ds(0, l3), :] = jnp.maximum(acc + bias3_ref[...], 0.0)
    a3_ref[pl.ds(l3, 4), :] = jnp.zeros((4, 512), f32)

    e3 = jnp.maximum(a3_ref[pl.ds(0, nb * 9 - 1), :],
                     a3_ref[pl.ds(1, nb * 9 - 1), :])
    e3_ref[pl.ds(0, nb * 9 - 1), :] = jnp.maximum(e3, _wshift(e3)).astype(bf16)
    e3_ref[pl.ds(nb * 9 - 1, 1), :] = jnp.zeros((1, 512), bf16)
    c3 = jnp.dot(s3_ref[...], e3_ref[...], preferred_element_type=f32)
    c3c = jnp.dot(c3.astype(bf16), c3m_ref[...],
                  preferred_element_type=f32)
    feats = jnp.concatenate([c3c[0:nb], c3c[nb:2 * nb]], axis=1).astype(bf16)

    h = jnp.dot(feats, fw1_ref[...], preferred_element_type=f32)
    h = jnp.maximum(h + fb1_ref[...], 0.0)
    logits = jnp.dot(h, fw2_ref[...], preferred_element_type=f32) + fb2_ref[...]
    m = jnp.max(logits, axis=-1, keepdims=True)
    ex = jnp.exp(logits - m)
    out_ref[...] = ex / jnp.sum(ex, axis=-1, keepdims=True)


def _toeplitz(w, hi, wo):
    kw = w.shape[1]
    shift = jnp.stack([jnp.eye(hi, wo, k=-x, dtype=w.dtype) for x in range(kw)])
    return jnp.einsum('xvw,hxcd->hcvdw', shift, w)


def _pad_k(t, kpad):
    five, k, n = t.shape
    z = jnp.zeros((five, kpad, n), t.dtype)
    return z.at[:, :k, :].set(t).reshape(five * kpad, n)


def _rowsel(nb, pitch_in, n_out):
    p = jnp.zeros((n_out, pitch_in), jnp.float32)
    p = p.at[jnp.arange(n_out), 2 * jnp.arange(n_out)].set(1.0)
    return jnp.kron(jnp.eye(nb, dtype=jnp.float32), p)


def _lanesel(c, w_in, w_out):
    p = jnp.zeros((w_in, w_out), jnp.float32)
    p = p.at[2 * jnp.arange(w_out), jnp.arange(w_out)].set(1.0)
    return jnp.kron(jnp.eye(c, dtype=jnp.float32), p)


@jax.jit
def _forward(x_nchw, w1m, b1, w2m, b2, w3m, b3, fw1, fb1, fw2, fb2):
    f32 = jnp.float32
    bf16 = jnp.bfloat16
    bsz = x_nchw.shape[0]
    nb = _NB

    w1r = w1m.reshape(5, 5, 1, 32)
    w2r = w2m.reshape(5, 5, 32, 64)
    w3r = w3m.reshape(5, 5, 64, 128)

    t1 = _pad_k(_toeplitz(w1r, 50, 46).reshape(5, 50, 1472), 64).astype(bf16)
    t2 = _pad_k(_toeplitz(w2r, 23, 18).reshape(5, 736, 1152), 768).astype(bf16)
    t3 = _pad_k(_toeplitz(w3r, 9, 4).reshape(5, 576, 512), 640).astype(bf16)

    fw1v = fw1.reshape(2, 2, 128, 512)
    fw1p = jnp.transpose(fw1v, (0, 2, 1, 3)).reshape(512, 512).astype(bf16)

    s1 = _rowsel(nb, 50, 25).astype(bf16)
    s2 = _rowsel(nb, 25, 9).astype(bf16)
    e0 = jnp.zeros((1, 9), f32).at[0, 0].set(1.0)
    e2v = jnp.zeros((1, 9), f32).at[0, 2].set(1.0)
    s3 = jnp.concatenate([jnp.kron(jnp.eye(nb, dtype=f32), e0),
                          jnp.kron(jnp.eye(nb, dtype=f32), e2v)], axis=0)
    s3 = s3.astype(bf16)

    c1 = _lanesel(32, 46, 23).astype(bf16)
    c2 = _lanesel(64, 18, 9).astype(bf16)
    c3m = _lanesel(128, 4, 2).astype(bf16)

    bias1 = jnp.repeat(b1.reshape(32, 1), 46, axis=1).reshape(1, 1472)
    bias2 = jnp.repeat(b2.reshape(64, 1), 18, axis=1).reshape(1, 1152)
    bias3 = jnp.repeat(b3.reshape(128, 1), 4, axis=1).reshape(1, 512)

    bpad = (-bsz) % nb
    x = x_nchw.astype(f32).reshape(bsz, 50, 50)
    if bpad:
        x = jnp.concatenate([x, jnp.zeros((bpad, 50, 50), f32)], axis=0)
    bt = bsz + bpad
    x = x.reshape(bt * 50, 50)

    const = lambda i: (0, 0)
    in_specs = [
        pl.BlockSpec((nb * 50, 50), lambda i: (i, 0)),
        pl.BlockSpec((5 * 64, 1472), const),
        pl.BlockSpec((5 * 768, 1152), const),
        pl.BlockSpec((5 * 640, 512), const),
        pl.BlockSpec((512, 512), const),
        pl.BlockSpec((nb * 25, nb * 50), const),
        pl.BlockSpec((nb * 9, nb * 25), const),
        pl.BlockSpec((2 * nb, nb * 9), const),
        pl.BlockSpec((1472, 736), const),
        pl.BlockSpec((1152, 576), const),
        pl.BlockSpec((512, 256), const),
        pl.BlockSpec((1, 1472), const),
        pl.BlockSpec((1, 1152), const),
        pl.BlockSpec((1, 512), const),
        pl.BlockSpec((1, 512), const),
        pl.BlockSpec((512, 2), const),
        pl.BlockSpec((1, 2), const),
    ]
    scratch_shapes = [
        pltpu.VMEM((nb * 50, 5 * 64), bf16),
        pltpu.VMEM((nb * 50, 1472), f32),
        pltpu.VMEM((nb * 50, 1472), bf16),
        pltpu.VMEM((nb * 25, 1472), bf16),
        pltpu.VMEM((nb * 25, 5 * 768), bf16),
        pltpu.VMEM((nb * 25, 1152), f32),
        pltpu.VMEM((nb * 25, 1152), bf16),
        pltpu.VMEM((nb * 9, 1152), bf16),
        pltpu.VMEM((nb * 9, 5 * 640), bf16),
        pltpu.VMEM((nb * 9, 512), f32),
        pltpu.VMEM((nb * 9, 512), bf16),
    ]
    cls = getattr(pltpu, "CompilerParams", None) or getattr(
        pltpu, "TPUCompilerParams", None)
    cparams = None
    if cls is not None:
        cparams = cls(dimension_semantics=("parallel",),
                      vmem_limit_bytes=56 * 1024 * 1024)

    out = pl.pallas_call(
        _cnn_body,
        out_shape=jax.ShapeDtypeStruct((bt, 2), f32),
        grid=(bt // nb,),
        in_specs=in_specs,
        out_specs=pl.BlockSpec((nb, 2), lambda i: (i, 0)),
        scratch_shapes=scratch_shapes,
        compiler_params=cparams,
    )(x, t1, t2, t3, fw1p, s1, s2, s3, c1, c2, c3m,
      bias1, bias2, bias3, fb1, fw2, fb2)
    return out[:bsz]


def kernel(x_nchw, w1m, b1, w2m, b2, w3m, b3, fw1, fb1, fw2, fb2):
    return _forward(x_nchw, w1m, b1, w2m, b2, w3m, b3, fw1, fb1, fw2, fb2)


# speedup vs baseline: 1.6650x; 1.0101x over previous
import jax
import jax.numpy as jnp
from jax.experimental import pallas as pl
from jax.experimental.pallas import tpu as pltpu

_NB = 16
_K5 = 5


def _wshift(v):
    return jnp.concatenate([v[:, 1:], v[:, :1]], axis=1)


def _cnn_body(x_ref, t1_ref, t2_ref, t3_ref, fw1_ref, s1_ref, s2_ref, s3_ref,
              c1_ref, c2_ref, c3m_ref,
              bias1_ref, bias2_ref, bias3_ref, fb1_ref, fw2_ref, fb2_ref,
              out_ref,
              x1p_ref, a1_ref, e1_ref, q1_ref, p1p_ref, a2_ref,
              e2_ref, q2_ref, p2p_ref, a3_ref, e3_ref):
    nb = _NB
    f32 = jnp.float32
    bf16 = jnp.bfloat16

    xv = x_ref[...].astype(bf16)
    l1 = nb * 50 - 4
    for kh in range(_K5):
        x1p_ref[pl.ds(0, l1), pl.ds(kh * 64, 50)] = xv[kh:kh + l1]
        x1p_ref[:, pl.ds(kh * 64 + 50, 14)] = jnp.zeros((nb * 50, 14), bf16)

    acc = jnp.dot(x1p_ref[pl.ds(0, l1), :], t1_ref[...],
                  preferred_element_type=f32)
    a1_ref[pl.ds(0, l1), :] = jnp.maximum(acc + bias1_ref[...],
                                          0.0).astype(bf16)
    a1_ref[pl.ds(l1, 4), :] = jnp.zeros((4, 1472), bf16)

    e1 = jnp.maximum(a1_ref[pl.ds(0, nb * 50 - 1), :],
                     a1_ref[pl.ds(1, nb * 50 - 1), :])
    e1_ref[pl.ds(0, nb * 50 - 1), :] = jnp.maximum(e1, _wshift(e1)).astype(bf16)
    e1_ref[pl.ds(nb * 50 - 1, 1), :] = jnp.zeros((1, 1472), bf16)
    q1_ref[...] = jnp.dot(s1_ref[...], e1_ref[...],
                          preferred_element_type=f32).astype(bf16)
    p1 = jnp.dot(q1_ref[...], c1_ref[...],
                 preferred_element_type=f32).astype(bf16)

    l2 = nb * 25 - 4
    for kh in range(_K5):
        p1p_ref[pl.ds(0, l2), pl.ds(kh * 768, 736)] = p1[kh:kh + l2]
        p1p_ref[:, pl.ds(kh * 768 + 736, 32)] = jnp.zeros((nb * 25, 32), bf16)

    acc = jnp.dot(p1p_ref[pl.ds(0, l2), :], t2_ref[...],
                  preferred_element_type=f32)
    a2_ref[pl.ds(0, l2), :] = jnp.maximum(acc + bias2_ref[...],
                                          0.0).astype(bf16)
    a2_ref[pl.ds(l2, 4), :] = jnp.zeros((4, 1152), bf16)

    e2 = jnp.maximum(a2_ref[pl.ds(0, nb * 25 - 1), :],
                     a2_ref[pl.ds(1, nb * 25 - 1), :])
    e2_ref[pl.ds(0, nb * 25 - 1), :] = jnp.maximum(e2, _wshift(e2)).astype(bf16)
    e2_ref[pl.ds(nb * 25 - 1, 1), :] = jnp.zeros((1, 1152), bf16)
    q2_ref[...] = jnp.dot(s2_ref[...], e2_ref[...],
                          preferred_element_type=f32).astype(bf16)
    p2 = jnp.dot(q2_ref[...], c2_ref[...],
                 preferred_element_type=f32).astype(bf16)

    l3 = nb * 9 - 4
    for kh in range(_K5):
        p2p_ref[pl.ds(0, l3), pl.ds(kh * 640, 576)] = p2[kh:kh + l3]
        p2p_ref[:, pl.ds(kh * 640 + 576, 64)] = jnp.zeros((nb * 9, 64), bf16)

    acc = jnp.dot(p2p_ref[pl.ds(0, l3), :], t3_ref[...],
                  preferred_element_type=f32)
    a3_ref[pl.ds(0, l3), :] = jnp.maximum(acc + bias3_ref[...],
                                          0.0).astype(bf16)
    a3_ref[pl.ds(l3, 4), :] = jnp.zeros((4, 512), bf16)

    e3 = jnp.maximum(a3_ref[pl.ds(0, nb * 9 - 1), :],
                     a3_ref[pl.ds(1, nb * 9 - 1), :])
    e3_ref[pl.ds(0, nb * 9 - 1), :] = jnp.maximum(e3, _wshift(e3)).astype(bf16)
    e3_ref[pl.ds(nb * 9 - 1, 1), :] = jnp.zeros((1, 512), bf16)
    c3 = jnp.dot(s3_ref[...], e3_ref[...], preferred_element_type=f32)
    c3c = jnp.dot(c3.astype(bf16), c3m_ref[...],
                  preferred_element_type=f32)
    feats = jnp.concatenate([c3c[0:nb], c3c[nb:2 * nb]], axis=1).astype(bf16)

    h = jnp.dot(feats, fw1_ref[...], preferred_element_type=f32)
    h = jnp.maximum(h + fb1_ref[...], 0.0)
    logits = jnp.dot(h, fw2_ref[...], preferred_element_type=f32) + fb2_ref[...]
    m = jnp.max(logits, axis=-1, keepdims=True)
    ex = jnp.exp(logits - m)
    out_ref[...] = ex / jnp.sum(ex, axis=-1, keepdims=True)


def _toeplitz(w, hi, wo):
    kw = w.shape[1]
    shift = jnp.stack([jnp.eye(hi, wo, k=-x, dtype=w.dtype) for x in range(kw)])
    return jnp.einsum('xvw,hxcd->hcvdw', shift, w)


def _pad_k(t, kpad):
    five, k, n = t.shape
    z = jnp.zeros((five, kpad, n), t.dtype)
    return z.at[:, :k, :].set(t).reshape(five * kpad, n)


def _rowsel(nb, pitch_in, n_out):
    p = jnp.zeros((n_out, pitch_in), jnp.float32)
    p = p.at[jnp.arange(n_out), 2 * jnp.arange(n_out)].set(1.0)
    return jnp.kron(jnp.eye(nb, dtype=jnp.float32), p)


def _rowshift(n, n_valid):
    blocks = []
    for kh in range(_K5):
        r = jnp.arange(n)
        ok = (r < n_valid) & (r + kh < n)
        m = jnp.zeros((n, n), jnp.float32)
        m = m.at[r, jnp.minimum(r + kh, n - 1)].set(jnp.where(ok, 1.0, 0.0))
        blocks.append(m)
    return jnp.concatenate(blocks, axis=0)


def _lanesel(c, w_in, w_out):
    p = jnp.zeros((w_in, w_out), jnp.float32)
    p = p.at[2 * jnp.arange(w_out), jnp.arange(w_out)].set(1.0)
    return jnp.kron(jnp.eye(c, dtype=jnp.float32), p)


@jax.jit
def _forward(x_nchw, w1m, b1, w2m, b2, w3m, b3, fw1, fb1, fw2, fb2):
    f32 = jnp.float32
    bf16 = jnp.bfloat16
    bsz = x_nchw.shape[0]
    nb = _NB

    w1r = w1m.reshape(5, 5, 1, 32)
    w2r = w2m.reshape(5, 5, 32, 64)
    w3r = w3m.reshape(5, 5, 64, 128)

    t1 = _pad_k(_toeplitz(w1r, 50, 46).reshape(5, 50, 1472), 64).astype(bf16)
    t2 = _pad_k(_toeplitz(w2r, 23, 18).reshape(5, 736, 1152), 768).astype(bf16)
    t3 = _pad_k(_toeplitz(w3r, 9, 4).reshape(5, 576, 512), 640).astype(bf16)

    fw1v = fw1.reshape(2, 2, 128, 512)
    fw1p = jnp.transpose(fw1v, (0, 2, 1, 3)).reshape(512, 512).astype(bf16)

    s1 = _rowsel(nb, 50, 25).astype(bf16)
    s2 = _rowsel(nb, 25, 9).astype(bf16)
    e0 = jnp.zeros((1, 9), f32).at[0, 0].set(1.0)
    e2v = jnp.zeros((1, 9), f32).at[0, 2].set(1.0)
    s3 = jnp.concatenate([jnp.kron(jnp.eye(nb, dtype=f32), e0),
                          jnp.kron(jnp.eye(nb, dtype=f32), e2v)], axis=0)
    s3 = s3.astype(bf16)

    c1 = _lanesel(32, 46, 23).astype(bf16)
    c2 = _lanesel(64, 18, 9).astype(bf16)
    c3m = _lanesel(128, 4, 2).astype(bf16)

    bias1 = jnp.repeat(b1.reshape(32, 1), 46, axis=1).reshape(1, 1472)
    bias2 = jnp.repeat(b2.reshape(64, 1), 18, axis=1).reshape(1, 1152)
    bias3 = jnp.repeat(b3.reshape(128, 1), 4, axis=1).reshape(1, 512)

    bpad = (-bsz) % nb
    x = x_nchw.astype(f32).reshape(bsz, 50, 50)
    if bpad:
        x = jnp.concatenate([x, jnp.zeros((bpad, 50, 50), f32)], axis=0)
    bt = bsz + bpad
    x = x.reshape(bt * 50, 50)

    const = lambda i: (0, 0)
    in_specs = [
        pl.BlockSpec((nb * 50, 50), lambda i: (i, 0)),
        pl.BlockSpec((5 * 64, 1472), const),
        pl.BlockSpec((5 * 768, 1152), const),
        pl.BlockSpec((5 * 640, 512), const),
        pl.BlockSpec((512, 512), const),
        pl.BlockSpec((nb * 25, nb * 50), const),
        pl.BlockSpec((nb * 9, nb * 25), const),
        pl.BlockSpec((2 * nb, nb * 9), const),
        pl.BlockSpec((1472, 736), const),
        pl.BlockSpec((1152, 576), const),
        pl.BlockSpec((512, 256), const),
        pl.BlockSpec((1, 1472), const),
        pl.BlockSpec((1, 1152), const),
        pl.BlockSpec((1, 512), const),
        pl.BlockSpec((1, 512), const),
        pl.BlockSpec((512, 2), const),
        pl.BlockSpec((1, 2), const),
    ]
    scratch_shapes = [
        pltpu.VMEM((nb * 50, 5 * 64), bf16),
        pltpu.VMEM((nb * 50, 1472), bf16),
        pltpu.VMEM((nb * 50, 1472), bf16),
        pltpu.VMEM((nb * 25, 1472), bf16),
        pltpu.VMEM((nb * 25, 5 * 768), bf16),
        pltpu.VMEM((nb * 25, 1152), bf16),
        pltpu.VMEM((nb * 25, 1152), bf16),
        pltpu.VMEM((nb * 9, 1152), bf16),
        pltpu.VMEM((nb * 9, 5 * 640), bf16),
        pltpu.VMEM((nb * 9, 512), bf16),
        pltpu.VMEM((nb * 9, 512), bf16),
    ]
    cls = getattr(pltpu, "CompilerParams", None) or getattr(
        pltpu, "TPUCompilerParams", None)
    cparams = None
    if cls is not None:
        cparams = cls(dimension_semantics=("parallel",),
                      vmem_limit_bytes=56 * 1024 * 1024)

    out = pl.pallas_call(
        _cnn_body,
        out_shape=jax.ShapeDtypeStruct((bt, 2), f32),
        grid=(bt // nb,),
        in_specs=in_specs,
        out_specs=pl.BlockSpec((nb, 2), lambda i: (i, 0)),
        scratch_shapes=scratch_shapes,
        compiler_params=cparams,
    )(x, t1, t2, t3, fw1p, s1, s2, s3, c1, c2, c3m,
      bias1, bias2, bias3, fb1, fw2, fb2)
    return out[:bsz]


def kernel(x_nchw, w1m, b1, w2m, b2, w3m, b3, fw1, fb1, fw2, fb2):
    return _forward(x_nchw, w1m, b1, w2m, b2, w3m, b3, fw1, fb1, fw2, fb2)
